# baseline (speedup 1.0000x reference)
import functools
import math

import jax
import jax.numpy as jnp
import numpy as np
from jax import lax
from jax.experimental import pallas as pl
from jax.experimental.pallas import tpu as pltpu
from jax.experimental.pallas import tpu_sc as plsc

B = 32
N = 96 * 56 * 56
K = math.ceil(0.5 * N)
NC, NS, L = 2, 16, 16

_SIGN = np.int32(-2147483648)


def _find_group(hist_ref, rank, total0):
    def cond(carry):
        g, _, found, _, _ = carry
        return jnp.logical_and(found == 0, g < 256)

    def body(carry):
        g, total, found, grp, cnt_before = carry
        acc = hist_ref[pl.ds(g * 256, L)]
        for j in range(1, 16):
            acc = acc + hist_ref[pl.ds(g * 256 + j * L, L)]
        s = jnp.sum(acc)
        hit = total + s >= rank
        grp = jnp.where(hit, g, grp)
        cnt_before = jnp.where(hit, total, cnt_before)
        found = jnp.where(hit, jnp.int32(1), found)
        return g + 1, total + s, found, grp, cnt_before

    init = (jnp.int32(0), total0, jnp.int32(0), jnp.int32(0), jnp.int32(0))
    _, _, _, grp, cnt_before = lax.while_loop(cond, body, init)
    return grp, cnt_before


def _find16(hist_ref, base, rank, total0):
    def body(j, carry):
        total, found, bin_idx, cnt_before = carry
        v = hist_ref[pl.ds(base + j * L, L)]
        s = jnp.sum(v)
        cs = plsc.cumsum(v)
        hit = jnp.logical_and(found == 0, total + s >= rank)
        below = (total + cs) < rank
        nbelow = jnp.max(plsc.all_reduce_population_count(below))
        cb = total + jnp.sum(jnp.where(below, v, 0))
        bin_idx = jnp.where(hit, j * L + nbelow, bin_idx)
        cnt_before = jnp.where(hit, cb, cnt_before)
        found = jnp.where(hit, jnp.int32(1), found)
        return total + s, found, bin_idx, cnt_before

    init = (total0, jnp.int32(0), jnp.int32(0), jnp.int32(0))
    _, _, bin_idx, cnt_before = lax.fori_loop(0, 16, body, init)
    return bin_idx, cnt_before


def _fused_sc(d4, w16):
    mesh = plsc.VectorSubcoreMesh(
        core_axis_name="c", subcore_axis_name="s", num_cores=NC, num_subcores=NS
    )
    P = 4
    NCH4 = 56 // P

    @functools.partial(
        pl.kernel,
        mesh=mesh,
        out_type=jax.ShapeDtypeStruct((B, 56, 56, 96), jnp.float32),
        compiler_params=pltpu.CompilerParams(
            needs_layout_passes=False, use_tc_tiling_on_sc=True
        ),
        scratch_types=[
            pltpu.VMEM((65536,), jnp.int32),
            pltpu.VMEM((P, 56, 96), jnp.float32),
            pltpu.VMEM((P, 56, 96), jnp.float32),
            pltpu.VMEM((L,), jnp.float32),
            pltpu.SemaphoreType.DMA,
            pltpu.SemaphoreType.DMA,
            pltpu.SemaphoreType.DMA,
            pltpu.SemaphoreType.DMA,
        ],
    )
    def sel(d_hbm, w_hbm, out_hbm, hist, buf0, buf1, wbuf, sem0, sem1, osem0, osem1):
        row = lax.axis_index("s") * NC + lax.axis_index("c")
        bufs = (buf0, buf1)
        sems = (sem0, sem1)
        osems = (osem0, osem1)
        ones = jnp.ones((L,), jnp.int32)
        zeros = jnp.zeros((L,), jnp.int32)
        pltpu.sync_copy(w_hbm, wbuf)
        wv = wbuf[...]

        def zero_hist():
            @plsc.parallel_loop(0, 65536, L, unroll=8)
            def _(j):
                hist[pl.ds(j, L)] = zeros

        def monotone(x):
            xi = lax.bitcast_convert_type(x, jnp.int32)
            s = lax.shift_right_arithmetic(xi, 31)
            return lax.bitwise_xor(xi, lax.bitwise_or(s, _SIGN))

        def pass1_vec(x):
            u = monotone(x)
            plsc.addupdate_scatter(hist, [lax.shift_right_logical(u, 16)], ones)

        def pass2_vec(x, b16):
            u = monotone(x)
            m = lax.shift_right_logical(u, 16) == b16
            lo = jnp.bitwise_and(u, 65535)
            plsc.addupdate_scatter(hist, [lo], ones, mask=m)

        last = jnp.int32(NCH4 - 1)

        def startin(c, i):
            pltpu.async_copy(d_hbm.at[row, pl.ds(c * P, P)], bufs[i], sems[i])

        def waitin(i):
            pltpu.make_async_copy(
                d_hbm.at[row, pl.ds(0, P)], bufs[i], sems[i]
            ).wait()

        def prime():
            startin(0, 0)
            startin(1, 1)

        def stream_pass(vec_fn):
            def process(b):
                @plsc.parallel_loop(0, 56, 1)
                def _(q):
                    for p in range(P):
                        for v in range(6):
                            vec_fn(b[p, q, pl.ds(v * L, L)])

            zero_hist()

            def body(it, carry):
                c0 = it * 2
                waitin(0)
                process(bufs[0])
                startin(jnp.minimum(c0 + 2, last), 0)
                waitin(1)
                process(bufs[1])
                startin(jnp.minimum(c0 + 3, last), 1)
                return carry

            lax.fori_loop(0, NCH4 // 2, body, 0)
            waitin(0)
            waitin(1)

        prime()
        stream_pass(pass1_vec)
        prime()
        bhi, cb = _find_group(hist, jnp.int32(K), jnp.int32(0))
        b16, cb2 = _find16(hist, bhi * 256, jnp.int32(K), cb)
        b16 = bhi * 256 + b16

        stream_pass(lambda x: pass2_vec(x, b16))
        prime()

        rank2 = jnp.int32(K) - cb2
        blo_hi, cb3 = _find_group(hist, rank2, jnp.int32(0))
        blo, _ = _find16(hist, blo_hi * 256, rank2, cb3)
        blo = blo_hi * 256 + blo

        thr_u = jnp.bitwise_or(lax.shift_left(b16, 16), blo)
        orig = jnp.where(
            thr_u < 0,
            lax.bitwise_xor(thr_u, _SIGN),
            jnp.bitwise_not(thr_u),
        )
        thr_vec = lax.bitcast_convert_type(
            jnp.broadcast_to(orig, (L,)), jnp.float32
        )
        tvec = jnp.minimum(thr_vec, wv)

        def process3(b):
            @plsc.parallel_loop(0, 56, 1)
            def _(q):
                for p in range(P):
                    for v in range(6):
                        x = b[p, q, pl.ds(v * L, L)]
                        r = jnp.where(x > tvec, wv, x)
                        b[p, q, pl.ds(v * L, L)] = 1.0 - r / wv

        def startout(c, i):
            pltpu.async_copy(
                bufs[i], out_hbm.at[row, pl.ds(c * P, P)], osems[i]
            )

        def waitout(i):
            pltpu.make_async_copy(
                bufs[i], out_hbm.at[row, pl.ds(0, P)], osems[i]
            ).wait()

        def body3(it, carry):
            c0 = it * 2
            waitin(0)
            process3(bufs[0])
            startout(c0, 0)
            waitin(1)
            process3(bufs[1])
            startout(c0 + 1, 1)
            waitout(0)
            startin(jnp.minimum(c0 + 2, last), 0)
            waitout(1)
            startin(jnp.minimum(c0 + 3, last), 1)
            return carry

        lax.fori_loop(0, NCH4 // 2, body3, 0)
        waitin(0)
        waitin(1)

    return sel(d4, w16)


def kernel(d, w):
    d_perm = jnp.transpose(d, (0, 2, 3, 1))
    w16 = jnp.broadcast_to(w, (L,))
    out_perm = _fused_sc(d_perm, w16)
    return jnp.transpose(out_perm, (0, 3, 1, 2))
